# baseline (speedup 1.0000x reference)
import functools

import jax
import jax.numpy as jnp
from jax import lax
from jax.experimental import pallas as pl
from jax.experimental.pallas import tpu as pltpu
from jax.experimental.pallas import tpu_sc as plsc

BATCH = 16384
EMBED_DIM = 64
_NC = 2
_NS = 16
_NW = _NC * _NS
_BPW = BATCH // _NW
_CHUNK = 128
_NCHUNK = _BPW // _CHUNK


def _body(uid_hbm, pid_hbm, nid_hbm, uemb_hbm, iemb_hbm,
          pos_hbm, neg_hbm,
          uidx_v, pidx_v, nidx_v, urows_v, prows_v, nrows_v,
          pout_v, nout_v, sem_u, sem_p, sem_n):
    wid = lax.axis_index("s") * _NC + lax.axis_index("c")

    pltpu.sync_copy(uid_hbm.at[wid], uidx_v)
    pltpu.sync_copy(pid_hbm.at[wid], pidx_v)
    pltpu.sync_copy(nid_hbm.at[wid], nidx_v)

    copies = []
    for j in range(_NCHUNK):
        dst = pl.ds(j * _CHUNK, _CHUNK)
        copies.append(pltpu.async_copy(uemb_hbm.at[uidx_v.at[j]],
                                       urows_v.at[dst], sem_u))
        copies.append(pltpu.async_copy(iemb_hbm.at[pidx_v.at[j]],
                                       prows_v.at[dst], sem_p))
        copies.append(pltpu.async_copy(iemb_hbm.at[nidx_v.at[j]],
                                       nrows_v.at[dst], sem_n))
    for c in copies:
        c.wait()

    lanes = lax.broadcasted_iota(jnp.int32, (16,), 0)

    def group(g, _):
        rvec = g * 16 + lanes
        accp = jnp.zeros((16,), jnp.float32)
        accn = jnp.zeros((16,), jnp.float32)
        for d in range(EMBED_DIM):
            dvec = jnp.full((16,), d, jnp.int32)
            u = plsc.load_gather(urows_v, [rvec, dvec])
            p = plsc.load_gather(prows_v, [rvec, dvec])
            n = plsc.load_gather(nrows_v, [rvec, dvec])
            accp = accp + u * p
            accn = accn + u * n
        pout_v[pl.ds(g * 16, 16)] = accp
        nout_v[pl.ds(g * 16, 16)] = accn
        return 0

    lax.fori_loop(0, _BPW // 16, group, 0)

    base = wid * _BPW
    pltpu.sync_copy(pout_v, pos_hbm.at[pl.ds(base, _BPW)])
    pltpu.sync_copy(nout_v, neg_hbm.at[pl.ds(base, _BPW)])


@jax.jit
def kernel(user_ids, pos_items, neg_items, user_emb, item_emb):
    mesh = plsc.VectorSubcoreMesh(core_axis_name="c", subcore_axis_name="s")
    f32 = jnp.float32
    run = pl.kernel(
        _body,
        out_type=(jax.ShapeDtypeStruct((BATCH,), f32),
                  jax.ShapeDtypeStruct((BATCH,), f32)),
        mesh=mesh,
        compiler_params=pltpu.CompilerParams(needs_layout_passes=False,
                                             use_tc_tiling_on_sc=False),
        scratch_types=[
            pltpu.VMEM((_NCHUNK, _CHUNK), jnp.int32),
            pltpu.VMEM((_NCHUNK, _CHUNK), jnp.int32),
            pltpu.VMEM((_NCHUNK, _CHUNK), jnp.int32),
            pltpu.VMEM((_BPW, EMBED_DIM), f32),
            pltpu.VMEM((_BPW, EMBED_DIM), f32),
            pltpu.VMEM((_BPW, EMBED_DIM), f32),
            pltpu.VMEM((_BPW,), f32),
            pltpu.VMEM((_BPW,), f32),
            pltpu.SemaphoreType.DMA,
            pltpu.SemaphoreType.DMA,
            pltpu.SemaphoreType.DMA,
        ],
    )
    uid = user_ids.astype(jnp.int32).reshape(_NW, _NCHUNK, _CHUNK)
    pid = pos_items.astype(jnp.int32).reshape(_NW, _NCHUNK, _CHUNK)
    nid = neg_items.astype(jnp.int32).reshape(_NW, _NCHUNK, _CHUNK)
    pos_scores, neg_scores = run(uid, pid, nid, user_emb, item_emb)
    return pos_scores, neg_scores


# speedup vs baseline: 1.0043x; 1.0043x over previous
import jax
import jax.numpy as jnp
from jax import lax
from jax.experimental import pallas as pl
from jax.experimental.pallas import tpu as pltpu
from jax.experimental.pallas import tpu_sc as plsc

BATCH = 16384
EMBED_DIM = 64
_NC = 2
_NS = 16
_NW = _NC * _NS
_BPW = BATCH // _NW
_CHUNK = 128
_NCHUNK = _BPW // _CHUNK


def _body(urow_hbm, prow_hbm, nrow_hbm, ucb_hbm, pcb_hbm, ncb_hbm,
          ue_hbm, ie_hbm,
          pos_hbm, neg_hbm,
          uidx_v, pidx_v, nidx_v, ucb_v, pcb_v, ncb_v,
          ubuf0, ubuf1, pbuf0, pbuf1, nbuf0, nbuf1,
          pout_v, nout_v, sem_u, sem_p, sem_n):
    wid = lax.axis_index("s") * _NC + lax.axis_index("c")

    pltpu.sync_copy(urow_hbm.at[wid], uidx_v)
    pltpu.sync_copy(prow_hbm.at[wid], pidx_v)
    pltpu.sync_copy(nrow_hbm.at[wid], nidx_v)
    pltpu.sync_copy(ucb_hbm.at[wid], ucb_v)
    pltpu.sync_copy(pcb_hbm.at[wid], pcb_v)
    pltpu.sync_copy(ncb_hbm.at[wid], ncb_v)

    ubufs = (ubuf0, ubuf1)
    pbufs = (pbuf0, pbuf1)
    nbufs = (nbuf0, nbuf1)

    def fire(j):
        b = j % 2
        return (pltpu.async_copy(ue_hbm.at[uidx_v.at[j]], ubufs[b], sem_u),
                pltpu.async_copy(ie_hbm.at[pidx_v.at[j]], pbufs[b], sem_p),
                pltpu.async_copy(ie_hbm.at[nidx_v.at[j]], nbufs[b], sem_n))

    inflight = fire(0)
    lanes = lax.broadcasted_iota(jnp.int32, (16,), 0)

    for j in range(_NCHUNK):
        for c in inflight:
            c.wait()
        if j + 1 < _NCHUNK:
            inflight = fire(j + 1)
        b = j % 2
        ub, pb, nb = ubufs[b], pbufs[b], nbufs[b]

        def group(k, _):
            rvec = k * 16 + lanes
            ucb = ucb_v[j, pl.ds(k * 16, 16)]
            pcb = pcb_v[j, pl.ds(k * 16, 16)]
            ncb = ncb_v[j, pl.ds(k * 16, 16)]
            accp = jnp.zeros((16,), jnp.float32)
            accn = jnp.zeros((16,), jnp.float32)
            for d in range(EMBED_DIM):
                u = plsc.load_gather(ub, [rvec, ucb + d])
                p = plsc.load_gather(pb, [rvec, pcb + d])
                n = plsc.load_gather(nb, [rvec, ncb + d])
                accp = accp + u * p
                accn = accn + u * n
            pout_v[pl.ds(j * _CHUNK + k * 16, 16)] = accp
            nout_v[pl.ds(j * _CHUNK + k * 16, 16)] = accn
            return 0

        lax.fori_loop(0, _CHUNK // 16, group, 0)

    base = wid * _BPW
    pltpu.sync_copy(pout_v, pos_hbm.at[pl.ds(base, _BPW)])
    pltpu.sync_copy(nout_v, neg_hbm.at[pl.ds(base, _BPW)])


@jax.jit
def kernel(user_ids, pos_items, neg_items, user_emb, item_emb):
    mesh = plsc.VectorSubcoreMesh(core_axis_name="c", subcore_axis_name="s")
    f32 = jnp.float32
    i32 = jnp.int32
    run = pl.kernel(
        _body,
        out_type=(jax.ShapeDtypeStruct((BATCH,), f32),
                  jax.ShapeDtypeStruct((BATCH,), f32)),
        mesh=mesh,
        compiler_params=pltpu.CompilerParams(needs_layout_passes=False,
                                             use_tc_tiling_on_sc=True),
        scratch_types=[
            pltpu.VMEM((_NCHUNK, _CHUNK), i32),
            pltpu.VMEM((_NCHUNK, _CHUNK), i32),
            pltpu.VMEM((_NCHUNK, _CHUNK), i32),
            pltpu.VMEM((_NCHUNK, _CHUNK), i32),
            pltpu.VMEM((_NCHUNK, _CHUNK), i32),
            pltpu.VMEM((_NCHUNK, _CHUNK), i32),
            pltpu.VMEM((_CHUNK, 2 * EMBED_DIM), f32),
            pltpu.VMEM((_CHUNK, 2 * EMBED_DIM), f32),
            pltpu.VMEM((_CHUNK, 2 * EMBED_DIM), f32),
            pltpu.VMEM((_CHUNK, 2 * EMBED_DIM), f32),
            pltpu.VMEM((_CHUNK, 2 * EMBED_DIM), f32),
            pltpu.VMEM((_CHUNK, 2 * EMBED_DIM), f32),
            pltpu.VMEM((_BPW,), f32),
            pltpu.VMEM((_BPW,), f32),
            pltpu.SemaphoreType.DMA,
            pltpu.SemaphoreType.DMA,
            pltpu.SemaphoreType.DMA,
        ],
    )
    uid = user_ids.astype(i32)
    pid = pos_items.astype(i32)
    nid = neg_items.astype(i32)
    shape3 = (_NW, _NCHUNK, _CHUNK)
    pos_scores, neg_scores = run(
        (uid >> 1).reshape(shape3), (pid >> 1).reshape(shape3),
        (nid >> 1).reshape(shape3),
        ((uid & 1) << 6).reshape(shape3), ((pid & 1) << 6).reshape(shape3),
        ((nid & 1) << 6).reshape(shape3),
        user_emb.reshape(500000, 2 * EMBED_DIM),
        item_emb.reshape(500000, 2 * EMBED_DIM),
    )
    return pos_scores, neg_scores


# speedup vs baseline: 1.9693x; 1.9608x over previous
import jax
import jax.numpy as jnp
from jax import lax
from jax.experimental import pallas as pl
from jax.experimental.pallas import tpu as pltpu
from jax.experimental.pallas import tpu_sc as plsc

BATCH = 16384
EMBED_DIM = 64
_NC = 2
_NS = 16
_NW = _NC * _NS
_BPW = BATCH // _NW
_CHUNK = 128
_NCHUNK = _BPW // _CHUNK

_NROW = 1000000
_UB = 4096
_HOFF = 122 * _UB
_PROWS = _NROW - _HOFF
_PGRID = (_PROWS + _UB - 1) // _UB


def _repack_body(lo_ref, hi_ref, out_ref):
    out_ref[:, 0:EMBED_DIM] = jnp.swapaxes(lo_ref[...], 0, 1)
    out_ref[:, EMBED_DIM:2 * EMBED_DIM] = jnp.swapaxes(hi_ref[...], 0, 1)


def _tc_repack(table_t):
    return pl.pallas_call(
        _repack_body,
        out_shape=jax.ShapeDtypeStruct((_PROWS, 2 * EMBED_DIM), jnp.float32),
        grid=(_PGRID,),
        in_specs=[
            pl.BlockSpec((EMBED_DIM, _UB), lambda g: (0, g)),
            pl.BlockSpec((EMBED_DIM, _UB), lambda g: (0, g + _HOFF // _UB)),
        ],
        out_specs=pl.BlockSpec((_UB, 2 * EMBED_DIM), lambda g: (g, 0)),
    )(table_t, table_t)


def _body(urow_hbm, prow_hbm, nrow_hbm, ucb_hbm, pcb_hbm, ncb_hbm,
          ue_hbm, ie_hbm,
          pos_hbm, neg_hbm,
          uidx_v, pidx_v, nidx_v, ucb_v, pcb_v, ncb_v,
          ubuf0, ubuf1, pbuf0, pbuf1, nbuf0, nbuf1,
          pout_v, nout_v, sem_u, sem_p, sem_n):
    wid = lax.axis_index("s") * _NC + lax.axis_index("c")

    pltpu.sync_copy(urow_hbm.at[wid], uidx_v)
    pltpu.sync_copy(prow_hbm.at[wid], pidx_v)
    pltpu.sync_copy(nrow_hbm.at[wid], nidx_v)
    pltpu.sync_copy(ucb_hbm.at[wid], ucb_v)
    pltpu.sync_copy(pcb_hbm.at[wid], pcb_v)
    pltpu.sync_copy(ncb_hbm.at[wid], ncb_v)

    ubufs = (ubuf0, ubuf1)
    pbufs = (pbuf0, pbuf1)
    nbufs = (nbuf0, nbuf1)

    def fire(j):
        b = j % 2
        return (pltpu.async_copy(ue_hbm.at[uidx_v.at[j]], ubufs[b], sem_u),
                pltpu.async_copy(ie_hbm.at[pidx_v.at[j]], pbufs[b], sem_p),
                pltpu.async_copy(ie_hbm.at[nidx_v.at[j]], nbufs[b], sem_n))

    inflight = fire(0)
    lanes = lax.broadcasted_iota(jnp.int32, (16,), 0)

    for j in range(_NCHUNK):
        for c in inflight:
            c.wait()
        if j + 1 < _NCHUNK:
            inflight = fire(j + 1)
        b = j % 2
        ub, pb, nb = ubufs[b], pbufs[b], nbufs[b]

        def group(k, _):
            rvec = k * 16 + lanes
            ucb = ucb_v[j, pl.ds(k * 16, 16)]
            pcb = pcb_v[j, pl.ds(k * 16, 16)]
            ncb = ncb_v[j, pl.ds(k * 16, 16)]
            accp = jnp.zeros((16,), jnp.float32)
            accn = jnp.zeros((16,), jnp.float32)
            for d in range(EMBED_DIM):
                u = plsc.load_gather(ub, [rvec, ucb + d])
                p = plsc.load_gather(pb, [rvec, pcb + d])
                n = plsc.load_gather(nb, [rvec, ncb + d])
                accp = accp + u * p
                accn = accn + u * n
            pout_v[pl.ds(j * _CHUNK + k * 16, 16)] = accp
            nout_v[pl.ds(j * _CHUNK + k * 16, 16)] = accn
            return 0

        lax.fori_loop(0, _CHUNK // 16, group, 0)

    base = wid * _BPW
    pltpu.sync_copy(pout_v, pos_hbm.at[pl.ds(base, _BPW)])
    pltpu.sync_copy(nout_v, neg_hbm.at[pl.ds(base, _BPW)])


@jax.jit
def kernel(user_ids, pos_items, neg_items, user_emb, item_emb):
    mesh = plsc.VectorSubcoreMesh(core_axis_name="c", subcore_axis_name="s")
    f32 = jnp.float32
    i32 = jnp.int32
    run = pl.kernel(
        _body,
        out_type=(jax.ShapeDtypeStruct((BATCH,), f32),
                  jax.ShapeDtypeStruct((BATCH,), f32)),
        mesh=mesh,
        compiler_params=pltpu.CompilerParams(needs_layout_passes=False,
                                             use_tc_tiling_on_sc=True),
        scratch_types=[
            pltpu.VMEM((_NCHUNK, _CHUNK), i32),
            pltpu.VMEM((_NCHUNK, _CHUNK), i32),
            pltpu.VMEM((_NCHUNK, _CHUNK), i32),
            pltpu.VMEM((_NCHUNK, _CHUNK), i32),
            pltpu.VMEM((_NCHUNK, _CHUNK), i32),
            pltpu.VMEM((_NCHUNK, _CHUNK), i32),
            pltpu.VMEM((_CHUNK, 2 * EMBED_DIM), f32),
            pltpu.VMEM((_CHUNK, 2 * EMBED_DIM), f32),
            pltpu.VMEM((_CHUNK, 2 * EMBED_DIM), f32),
            pltpu.VMEM((_CHUNK, 2 * EMBED_DIM), f32),
            pltpu.VMEM((_CHUNK, 2 * EMBED_DIM), f32),
            pltpu.VMEM((_CHUNK, 2 * EMBED_DIM), f32),
            pltpu.VMEM((_BPW,), f32),
            pltpu.VMEM((_BPW,), f32),
            pltpu.SemaphoreType.DMA,
            pltpu.SemaphoreType.DMA,
            pltpu.SemaphoreType.DMA,
        ],
    )
    ue_p = _tc_repack(jnp.swapaxes(user_emb, 0, 1))
    ie_p = _tc_repack(jnp.swapaxes(item_emb, 0, 1))

    def split(idx):
        idx = idx.astype(i32)
        row = jnp.where(idx < _HOFF, idx, idx - _HOFF)
        cb = jnp.where(idx < _HOFF, 0, EMBED_DIM).astype(i32)
        shape3 = (_NW, _NCHUNK, _CHUNK)
        return row.reshape(shape3), cb.reshape(shape3)

    urow, ucb = split(user_ids)
    prow, pcb = split(pos_items)
    nrow, ncb = split(neg_items)
    pos_scores, neg_scores = run(urow, prow, nrow, ucb, pcb, ncb, ue_p, ie_p)
    return pos_scores, neg_scores


# speedup vs baseline: 2.2233x; 1.1290x over previous
import jax
import jax.numpy as jnp
from jax import lax
from jax.experimental import pallas as pl
from jax.experimental.pallas import tpu as pltpu
from jax.experimental.pallas import tpu_sc as plsc

BATCH = 16384
EMBED_DIM = 64
_NC = 2
_NS = 16
_NW = _NC * _NS
_BPW = BATCH // _NW
_CHUNK = 128
_NCHUNK = _BPW // _CHUNK

_NROW = 1000000
_UB = 8192
_HOFF = 61 * _UB
_PROWS = _NROW - _HOFF
_PGRID = (_PROWS + _UB - 1) // _UB


def _repack_body(lo_ref, hi_ref, out_ref):
    out_ref[:, 0:EMBED_DIM] = jnp.swapaxes(lo_ref[...], 0, 1)
    out_ref[:, EMBED_DIM:2 * EMBED_DIM] = jnp.swapaxes(hi_ref[...], 0, 1)


def _tc_repack(table_t):
    return pl.pallas_call(
        _repack_body,
        out_shape=jax.ShapeDtypeStruct((_PROWS, 2 * EMBED_DIM), jnp.float32),
        grid=(_PGRID,),
        in_specs=[
            pl.BlockSpec((EMBED_DIM, _UB), lambda g: (0, g)),
            pl.BlockSpec((EMBED_DIM, _UB), lambda g: (0, g + _HOFF // _UB)),
        ],
        out_specs=pl.BlockSpec((_UB, 2 * EMBED_DIM), lambda g: (g, 0)),
    )(table_t, table_t)


def _body(urow_hbm, prow_hbm, nrow_hbm, ucb_hbm, pcb_hbm, ncb_hbm,
          ue_hbm, ie_hbm,
          pos_hbm, neg_hbm,
          uidx_v, pidx_v, nidx_v, ucb_v, pcb_v, ncb_v,
          ubuf0, ubuf1, pbuf0, pbuf1, nbuf0, nbuf1,
          pout_v, nout_v, sem_u, sem_p, sem_n):
    wid = lax.axis_index("s") * _NC + lax.axis_index("c")

    pltpu.sync_copy(urow_hbm.at[wid], uidx_v)
    pltpu.sync_copy(prow_hbm.at[wid], pidx_v)
    pltpu.sync_copy(nrow_hbm.at[wid], nidx_v)
    pltpu.sync_copy(ucb_hbm.at[wid], ucb_v)
    pltpu.sync_copy(pcb_hbm.at[wid], pcb_v)
    pltpu.sync_copy(ncb_hbm.at[wid], ncb_v)

    ubufs = (ubuf0, ubuf1)
    pbufs = (pbuf0, pbuf1)
    nbufs = (nbuf0, nbuf1)

    def fire(j):
        b = j % 2
        return (pltpu.async_copy(ue_hbm.at[uidx_v.at[j]], ubufs[b], sem_u),
                pltpu.async_copy(ie_hbm.at[pidx_v.at[j]], pbufs[b], sem_p),
                pltpu.async_copy(ie_hbm.at[nidx_v.at[j]], nbufs[b], sem_n))

    inflight = fire(0)
    lanes = lax.broadcasted_iota(jnp.int32, (16,), 0)

    for j in range(_NCHUNK):
        for c in inflight:
            c.wait()
        if j + 1 < _NCHUNK:
            inflight = fire(j + 1)
        b = j % 2
        ub, pb, nb = ubufs[b], pbufs[b], nbufs[b]

        def group(k, _):
            rvec = k * 16 + lanes
            ucb = ucb_v[j, pl.ds(k * 16, 16)]
            pcb = pcb_v[j, pl.ds(k * 16, 16)]
            ncb = ncb_v[j, pl.ds(k * 16, 16)]
            accp = jnp.zeros((16,), jnp.float32)
            accn = jnp.zeros((16,), jnp.float32)
            for d in range(EMBED_DIM):
                u = plsc.load_gather(ub, [rvec, ucb + d])
                p = plsc.load_gather(pb, [rvec, pcb + d])
                n = plsc.load_gather(nb, [rvec, ncb + d])
                accp = accp + u * p
                accn = accn + u * n
            pout_v[pl.ds(j * _CHUNK + k * 16, 16)] = accp
            nout_v[pl.ds(j * _CHUNK + k * 16, 16)] = accn
            return 0

        lax.fori_loop(0, _CHUNK // 16, group, 0)

    base = wid * _BPW
    pltpu.sync_copy(pout_v, pos_hbm.at[pl.ds(base, _BPW)])
    pltpu.sync_copy(nout_v, neg_hbm.at[pl.ds(base, _BPW)])


@jax.jit
def kernel(user_ids, pos_items, neg_items, user_emb, item_emb):
    mesh = plsc.VectorSubcoreMesh(core_axis_name="c", subcore_axis_name="s")
    f32 = jnp.float32
    i32 = jnp.int32
    run = pl.kernel(
        _body,
        out_type=(jax.ShapeDtypeStruct((BATCH,), f32),
                  jax.ShapeDtypeStruct((BATCH,), f32)),
        mesh=mesh,
        compiler_params=pltpu.CompilerParams(needs_layout_passes=False,
                                             use_tc_tiling_on_sc=True),
        scratch_types=[
            pltpu.VMEM((_NCHUNK, _CHUNK), i32),
            pltpu.VMEM((_NCHUNK, _CHUNK), i32),
            pltpu.VMEM((_NCHUNK, _CHUNK), i32),
            pltpu.VMEM((_NCHUNK, _CHUNK), i32),
            pltpu.VMEM((_NCHUNK, _CHUNK), i32),
            pltpu.VMEM((_NCHUNK, _CHUNK), i32),
            pltpu.VMEM((_CHUNK, 2 * EMBED_DIM), f32),
            pltpu.VMEM((_CHUNK, 2 * EMBED_DIM), f32),
            pltpu.VMEM((_CHUNK, 2 * EMBED_DIM), f32),
            pltpu.VMEM((_CHUNK, 2 * EMBED_DIM), f32),
            pltpu.VMEM((_CHUNK, 2 * EMBED_DIM), f32),
            pltpu.VMEM((_CHUNK, 2 * EMBED_DIM), f32),
            pltpu.VMEM((_BPW,), f32),
            pltpu.VMEM((_BPW,), f32),
            pltpu.SemaphoreType.DMA,
            pltpu.SemaphoreType.DMA,
            pltpu.SemaphoreType.DMA,
        ],
    )
    ue_p = _tc_repack(jnp.swapaxes(user_emb, 0, 1))
    ie_p = _tc_repack(jnp.swapaxes(item_emb, 0, 1))

    def split(idx):
        idx = idx.astype(i32)
        row = jnp.where(idx < _HOFF, idx, idx - _HOFF)
        cb = jnp.where(idx < _HOFF, 0, EMBED_DIM).astype(i32)
        shape3 = (_NW, _NCHUNK, _CHUNK)
        return row.reshape(shape3), cb.reshape(shape3)

    urow, ucb = split(user_ids)
    prow, pcb = split(pos_items)
    nrow, ncb = split(neg_items)
    pos_scores, neg_scores = run(urow, prow, nrow, ucb, pcb, ncb, ue_p, ie_p)
    return pos_scores, neg_scores


# speedup vs baseline: 3.8030x; 1.7105x over previous
import jax
import jax.numpy as jnp
from jax import lax
from jax.experimental import pallas as pl
from jax.experimental.pallas import tpu as pltpu
from jax.experimental.pallas import tpu_sc as plsc

BATCH = 16384
EMBED_DIM = 64
_NC = 2
_NS = 16
_NW = _NC * _NS
_BPW = BATCH // _NW
_CHUNK = 128
_NCHUNK = _BPW // _CHUNK

_NROW = 1000000
_UB = 8192
_HOFF = 31 * _UB
_PROWS = _HOFF
_PGRID = _PROWS // _UB


def _bf16_bits(x):
    b = lax.bitcast_convert_type(x, jnp.int32)
    r = b + 0x7FFF + lax.bitwise_and(lax.shift_right_logical(b, 16), 1)
    return lax.bitwise_and(lax.shift_right_logical(r, 16), 0xFFFF)


def _repack_body(i0, i1, i2, i3, out_ref):
    ws = []
    for ref in (i0, i1, i2, i3):
        lo = _bf16_bits(ref[0:EMBED_DIM // 2, :])
        hi = _bf16_bits(ref[EMBED_DIM // 2:EMBED_DIM, :])
        ws.append(lax.bitwise_or(lo, lax.shift_left(hi, 16)))
    out_ref[...] = jnp.swapaxes(jnp.concatenate(ws, axis=0), 0, 1)


def _tc_repack(table_t):
    return pl.pallas_call(
        _repack_body,
        out_shape=jax.ShapeDtypeStruct((_PROWS, 2 * EMBED_DIM), jnp.int32),
        grid=(_PGRID,),
        in_specs=[
            pl.BlockSpec((EMBED_DIM, _UB), lambda g: (0, g)),
            pl.BlockSpec((EMBED_DIM, _UB), lambda g: (0, g + 31)),
            pl.BlockSpec((EMBED_DIM, _UB), lambda g: (0, g + 62)),
            pl.BlockSpec((EMBED_DIM, _UB),
                         lambda g: (0, jnp.minimum(g + 93, 122))),
        ],
        out_specs=pl.BlockSpec((_UB, 2 * EMBED_DIM), lambda g: (g, 0)),
    )(table_t, table_t, table_t, table_t)


def _body(urow_hbm, prow_hbm, nrow_hbm, ucb_hbm, pcb_hbm, ncb_hbm,
          ue_hbm, ie_hbm,
          pos_hbm, neg_hbm,
          uidx_v, pidx_v, nidx_v, ucb_v, pcb_v, ncb_v,
          ubuf0, ubuf1, pbuf0, pbuf1, nbuf0, nbuf1,
          pout_v, nout_v, sem_u, sem_p, sem_n):
    wid = lax.axis_index("s") * _NC + lax.axis_index("c")

    pltpu.sync_copy(urow_hbm.at[wid], uidx_v)
    pltpu.sync_copy(prow_hbm.at[wid], pidx_v)
    pltpu.sync_copy(nrow_hbm.at[wid], nidx_v)
    pltpu.sync_copy(ucb_hbm.at[wid], ucb_v)
    pltpu.sync_copy(pcb_hbm.at[wid], pcb_v)
    pltpu.sync_copy(ncb_hbm.at[wid], ncb_v)

    ubufs = (ubuf0, ubuf1)
    pbufs = (pbuf0, pbuf1)
    nbufs = (nbuf0, nbuf1)

    def fire(j):
        b = j % 2
        return (pltpu.async_copy(ue_hbm.at[uidx_v.at[j]], ubufs[b], sem_u),
                pltpu.async_copy(ie_hbm.at[pidx_v.at[j]], pbufs[b], sem_p),
                pltpu.async_copy(ie_hbm.at[nidx_v.at[j]], nbufs[b], sem_n))

    inflight = fire(0)
    lanes = lax.broadcasted_iota(jnp.int32, (16,), 0)
    himask = jnp.full((16,), -65536, jnp.int32)

    def unpack(g):
        lo = plsc.bitcast(lax.shift_left(g, 16), jnp.float32)
        hi = plsc.bitcast(lax.bitwise_and(g, himask), jnp.float32)
        return lo, hi

    for j in range(_NCHUNK):
        for c in inflight:
            c.wait()
        if j + 1 < _NCHUNK:
            inflight = fire(j + 1)
        b = j % 2
        ub, pb, nb = ubufs[b], pbufs[b], nbufs[b]

        def group(k, _):
            rvec = k * 16 + lanes
            ucb = ucb_v[j, pl.ds(k * 16, 16)]
            pcb = pcb_v[j, pl.ds(k * 16, 16)]
            ncb = ncb_v[j, pl.ds(k * 16, 16)]
            accp = jnp.zeros((16,), jnp.float32)
            accn = jnp.zeros((16,), jnp.float32)
            for pd in range(EMBED_DIM // 2):
                ulo, uhi = unpack(plsc.load_gather(ub, [rvec, ucb + pd]))
                plo, phi = unpack(plsc.load_gather(pb, [rvec, pcb + pd]))
                nlo, nhi = unpack(plsc.load_gather(nb, [rvec, ncb + pd]))
                accp = accp + ulo * plo + uhi * phi
                accn = accn + ulo * nlo + uhi * nhi
            pout_v[pl.ds(j * _CHUNK + k * 16, 16)] = accp
            nout_v[pl.ds(j * _CHUNK + k * 16, 16)] = accn
            return 0

        lax.fori_loop(0, _CHUNK // 16, group, 0)

    base = wid * _BPW
    pltpu.sync_copy(pout_v, pos_hbm.at[pl.ds(base, _BPW)])
    pltpu.sync_copy(nout_v, neg_hbm.at[pl.ds(base, _BPW)])


@jax.jit
def kernel(user_ids, pos_items, neg_items, user_emb, item_emb):
    mesh = plsc.VectorSubcoreMesh(core_axis_name="c", subcore_axis_name="s")
    f32 = jnp.float32
    i32 = jnp.int32
    run = pl.kernel(
        _body,
        out_type=(jax.ShapeDtypeStruct((BATCH,), f32),
                  jax.ShapeDtypeStruct((BATCH,), f32)),
        mesh=mesh,
        compiler_params=pltpu.CompilerParams(needs_layout_passes=False,
                                             use_tc_tiling_on_sc=True),
        scratch_types=[
            pltpu.VMEM((_NCHUNK, _CHUNK), i32),
            pltpu.VMEM((_NCHUNK, _CHUNK), i32),
            pltpu.VMEM((_NCHUNK, _CHUNK), i32),
            pltpu.VMEM((_NCHUNK, _CHUNK), i32),
            pltpu.VMEM((_NCHUNK, _CHUNK), i32),
            pltpu.VMEM((_NCHUNK, _CHUNK), i32),
            pltpu.VMEM((_CHUNK, 2 * EMBED_DIM), i32),
            pltpu.VMEM((_CHUNK, 2 * EMBED_DIM), i32),
            pltpu.VMEM((_CHUNK, 2 * EMBED_DIM), i32),
            pltpu.VMEM((_CHUNK, 2 * EMBED_DIM), i32),
            pltpu.VMEM((_CHUNK, 2 * EMBED_DIM), i32),
            pltpu.VMEM((_CHUNK, 2 * EMBED_DIM), i32),
            pltpu.VMEM((_BPW,), f32),
            pltpu.VMEM((_BPW,), f32),
            pltpu.SemaphoreType.DMA,
            pltpu.SemaphoreType.DMA,
            pltpu.SemaphoreType.DMA,
        ],
    )
    ue_p = _tc_repack(jnp.swapaxes(user_emb, 0, 1))
    ie_p = _tc_repack(jnp.swapaxes(item_emb, 0, 1))

    def split(idx):
        idx = idx.astype(i32)
        q = idx // _HOFF
        row = idx - q * _HOFF
        cb = q * 32
        shape3 = (_NW, _NCHUNK, _CHUNK)
        return row.reshape(shape3), cb.reshape(shape3)

    urow, ucb = split(user_ids)
    prow, pcb = split(pos_items)
    nrow, ncb = split(neg_items)
    pos_scores, neg_scores = run(urow, prow, nrow, ucb, pcb, ncb, ue_p, ie_p)
    return pos_scores, neg_scores


# speedup vs baseline: 3.8794x; 1.0201x over previous
import jax
import jax.numpy as jnp
from jax import lax
from jax.experimental import pallas as pl
from jax.experimental.pallas import tpu as pltpu
from jax.experimental.pallas import tpu_sc as plsc

BATCH = 16384
EMBED_DIM = 64
_NC = 2
_NS = 16
_NW = _NC * _NS
_BPW = BATCH // _NW
_CHUNK = 128
_NCHUNK = _BPW // _CHUNK

_NROW = 1000000
_UB = 16384
_HOFF = 16 * _UB
_PROWS = _HOFF
_PGRID = _PROWS // _UB


def _bf16_bits(x):
    b = lax.bitcast_convert_type(x, jnp.int32)
    r = b + 0x7FFF + lax.bitwise_and(lax.shift_right_logical(b, 16), 1)
    return lax.bitwise_and(lax.shift_right_logical(r, 16), 0xFFFF)


def _repack_body(i0, i1, i2, i3, out_ref):
    ws = []
    for ref in (i0, i1, i2, i3):
        lo = _bf16_bits(ref[0:EMBED_DIM // 2, :])
        hi = _bf16_bits(ref[EMBED_DIM // 2:EMBED_DIM, :])
        ws.append(lax.bitwise_or(lo, lax.shift_left(hi, 16)))
    out_ref[...] = jnp.swapaxes(jnp.concatenate(ws, axis=0), 0, 1)


def _tc_repack(table_t):
    return pl.pallas_call(
        _repack_body,
        out_shape=jax.ShapeDtypeStruct((_PROWS, 2 * EMBED_DIM), jnp.int32),
        grid=(_PGRID,),
        in_specs=[
            pl.BlockSpec((EMBED_DIM, _UB), lambda g: (0, g)),
            pl.BlockSpec((EMBED_DIM, _UB), lambda g: (0, g + 16)),
            pl.BlockSpec((EMBED_DIM, _UB), lambda g: (0, g + 32)),
            pl.BlockSpec((EMBED_DIM, _UB),
                         lambda g: (0, jnp.minimum(g + 48, 61))),
        ],
        out_specs=pl.BlockSpec((_UB, 2 * EMBED_DIM), lambda g: (g, 0)),
    )(table_t, table_t, table_t, table_t)


def _body(urow_hbm, prow_hbm, nrow_hbm, ucb_hbm, pcb_hbm, ncb_hbm,
          ue_hbm, ie_hbm,
          pos_hbm, neg_hbm,
          uidx_v, pidx_v, nidx_v, ucb_v, pcb_v, ncb_v,
          ubuf0, ubuf1, pbuf0, pbuf1, nbuf0, nbuf1,
          pout_v, nout_v, sem_u, sem_p, sem_n):
    wid = lax.axis_index("s") * _NC + lax.axis_index("c")

    pltpu.sync_copy(urow_hbm.at[wid], uidx_v)
    pltpu.sync_copy(prow_hbm.at[wid], pidx_v)
    pltpu.sync_copy(nrow_hbm.at[wid], nidx_v)
    pltpu.sync_copy(ucb_hbm.at[wid], ucb_v)
    pltpu.sync_copy(pcb_hbm.at[wid], pcb_v)
    pltpu.sync_copy(ncb_hbm.at[wid], ncb_v)

    ubufs = (ubuf0, ubuf1)
    pbufs = (pbuf0, pbuf1)
    nbufs = (nbuf0, nbuf1)

    def fire(j):
        b = j % 2
        return (pltpu.async_copy(ue_hbm.at[uidx_v.at[j]], ubufs[b], sem_u),
                pltpu.async_copy(ie_hbm.at[pidx_v.at[j]], pbufs[b], sem_p),
                pltpu.async_copy(ie_hbm.at[nidx_v.at[j]], nbufs[b], sem_n))

    inflight = fire(0)
    lanes = lax.broadcasted_iota(jnp.int32, (16,), 0)
    himask = jnp.full((16,), -65536, jnp.int32)

    def unpack(g):
        lo = plsc.bitcast(lax.shift_left(g, 16), jnp.float32)
        hi = plsc.bitcast(lax.bitwise_and(g, himask), jnp.float32)
        return lo, hi

    for j in range(_NCHUNK):
        for c in inflight:
            c.wait()
        if j + 1 < _NCHUNK:
            inflight = fire(j + 1)
        b = j % 2
        ub, pb, nb = ubufs[b], pbufs[b], nbufs[b]

        def group(k, _):
            rvec = k * 16 + lanes
            ucb = ucb_v[j, pl.ds(k * 16, 16)]
            pcb = pcb_v[j, pl.ds(k * 16, 16)]
            ncb = ncb_v[j, pl.ds(k * 16, 16)]
            accp = jnp.zeros((16,), jnp.float32)
            accn = jnp.zeros((16,), jnp.float32)
            for pd in range(EMBED_DIM // 2):
                ulo, uhi = unpack(plsc.load_gather(ub, [rvec, ucb + pd]))
                plo, phi = unpack(plsc.load_gather(pb, [rvec, pcb + pd]))
                nlo, nhi = unpack(plsc.load_gather(nb, [rvec, ncb + pd]))
                accp = accp + ulo * plo + uhi * phi
                accn = accn + ulo * nlo + uhi * nhi
            pout_v[pl.ds(j * _CHUNK + k * 16, 16)] = accp
            nout_v[pl.ds(j * _CHUNK + k * 16, 16)] = accn
            return 0

        lax.fori_loop(0, _CHUNK // 16, group, 0)

    base = wid * _BPW
    pltpu.sync_copy(pout_v, pos_hbm.at[pl.ds(base, _BPW)])
    pltpu.sync_copy(nout_v, neg_hbm.at[pl.ds(base, _BPW)])


@jax.jit
def kernel(user_ids, pos_items, neg_items, user_emb, item_emb):
    mesh = plsc.VectorSubcoreMesh(core_axis_name="c", subcore_axis_name="s")
    f32 = jnp.float32
    i32 = jnp.int32
    run = pl.kernel(
        _body,
        out_type=(jax.ShapeDtypeStruct((BATCH,), f32),
                  jax.ShapeDtypeStruct((BATCH,), f32)),
        mesh=mesh,
        compiler_params=pltpu.CompilerParams(needs_layout_passes=False,
                                             use_tc_tiling_on_sc=True),
        scratch_types=[
            pltpu.VMEM((_NCHUNK, _CHUNK), i32),
            pltpu.VMEM((_NCHUNK, _CHUNK), i32),
            pltpu.VMEM((_NCHUNK, _CHUNK), i32),
            pltpu.VMEM((_NCHUNK, _CHUNK), i32),
            pltpu.VMEM((_NCHUNK, _CHUNK), i32),
            pltpu.VMEM((_NCHUNK, _CHUNK), i32),
            pltpu.VMEM((_CHUNK, 2 * EMBED_DIM), i32),
            pltpu.VMEM((_CHUNK, 2 * EMBED_DIM), i32),
            pltpu.VMEM((_CHUNK, 2 * EMBED_DIM), i32),
            pltpu.VMEM((_CHUNK, 2 * EMBED_DIM), i32),
            pltpu.VMEM((_CHUNK, 2 * EMBED_DIM), i32),
            pltpu.VMEM((_CHUNK, 2 * EMBED_DIM), i32),
            pltpu.VMEM((_BPW,), f32),
            pltpu.VMEM((_BPW,), f32),
            pltpu.SemaphoreType.DMA,
            pltpu.SemaphoreType.DMA,
            pltpu.SemaphoreType.DMA,
        ],
    )
    ue_p = _tc_repack(jnp.swapaxes(user_emb, 0, 1))
    ie_p = _tc_repack(jnp.swapaxes(item_emb, 0, 1))

    def split(idx):
        idx = idx.astype(i32)
        q = idx // _HOFF
        row = idx - q * _HOFF
        cb = q * 32
        shape3 = (_NW, _NCHUNK, _CHUNK)
        return row.reshape(shape3), cb.reshape(shape3)

    urow, ucb = split(user_ids)
    prow, pcb = split(pos_items)
    nrow, ncb = split(neg_items)
    pos_scores, neg_scores = run(urow, prow, nrow, ucb, pcb, ncb, ue_p, ie_p)
    return pos_scores, neg_scores
